# baseline (speedup 1.0000x reference)
import functools

import jax
import jax.numpy as jnp
from jax.experimental import pallas as pl
from jax.experimental.pallas import tpu as pltpu

_MASK_RATIO = 0.5
_EPS = 1e-6


def _fused_body(y_ref, z_ref, m_sc, s_sc, acc, ent_sc, mask_sc, *,
                nb, nh, wg, k):
    p = pl.program_id(0)
    b = pl.program_id(1)
    h = pl.program_id(2)
    blk = y_ref[...]
    hb, w, c = blk.shape[1], blk.shape[2], blk.shape[3]

    @pl.when(p == 0)
    def _():
        pm = jnp.max(blk, axis=(0, 1, 2))
        prev = jnp.where(h == 0, jnp.full_like(pm, -jnp.inf), m_sc[b, :])
        m_sc[b, :] = jnp.maximum(prev, pm)

    @pl.when(p == 1)
    def _():
        m = m_sc[b, :]
        e = jnp.exp(blk - m[None, None, None, :])
        part = jnp.sum(e.reshape(hb * (w // wg), wg, c), axis=0)
        acc[...] = jnp.where(h == 0, part, acc[...] + part)

        @pl.when(h == nh - 1)
        def _():
            s_sc[b, :] = jnp.sum(acc[...], axis=0)

    @pl.when(p == 2)
    def _():
        m = m_sc[b, :]
        s = s_sc[b, :]
        e = jnp.exp(blk - m[None, None, None, :])
        q = e / s[None, None, None, :] + _EPS
        t = q * jnp.log(q)
        part = jnp.sum(t.reshape(hb * (w // wg), wg, c), axis=0)
        acc[...] = jnp.where(h == 0, part, acc[...] + part)

        @pl.when(h == nh - 1)
        def _():
            ent_sc[b, :] = -jnp.sum(acc[...], axis=0)

        @pl.when((b == nb - 1) & (h == nh - 1))
        def _():
            ent = ent_sc[...]
            ent_i = ent[:, :, None]
            ent_j = ent[:, None, :]
            lt = (ent_j < ent_i).astype(jnp.float32)
            eq = ent_j == ent_i
            jj = jax.lax.broadcasted_iota(jnp.int32, (nb, c, c), 2)
            ii = jax.lax.broadcasted_iota(jnp.int32, (nb, c, c), 1)
            tie = (eq & (jj < ii)).astype(jnp.float32)
            rank = jnp.sum(lt + tie, axis=2)
            mask_sc[...] = (rank < k).astype(jnp.float32)

    @pl.when(p == 3)
    def _():
        z_ref[...] = blk * mask_sc[b, :][None, None, None, :]


def kernel(x):
    B, C, H, W = x.shape
    k = int(C * (1 - _MASK_RATIO))
    y = jnp.transpose(x, (0, 2, 3, 1))
    HB = 28
    NH = H // HB
    WG = 8

    def zmap(p, b, h):
        sel = p == 3
        return (jnp.where(sel, b, 0), jnp.where(sel, h, 0), 0, 0)

    z = pl.pallas_call(
        functools.partial(_fused_body, nb=B, nh=NH, wg=WG, k=k),
        grid=(4, B, NH),
        in_specs=[pl.BlockSpec((1, HB, W, C), lambda p, b, h: (b, h, 0, 0))],
        out_specs=pl.BlockSpec((1, HB, W, C), zmap),
        out_shape=jax.ShapeDtypeStruct((B, H, W, C), jnp.float32),
        scratch_shapes=[
            pltpu.VMEM((B, C), jnp.float32),
            pltpu.VMEM((B, C), jnp.float32),
            pltpu.VMEM((WG, C), jnp.float32),
            pltpu.VMEM((B, C), jnp.float32),
            pltpu.VMEM((B, C), jnp.float32),
        ],
    )(y)

    return jnp.transpose(z, (0, 3, 1, 2))


# speedup vs baseline: 1.0205x; 1.0205x over previous
import functools

import jax
import jax.numpy as jnp
from jax import lax
from jax.experimental import pallas as pl
from jax.experimental.pallas import tpu as pltpu
from jax.experimental.pallas import tpu_sc as plsc

_MASK_RATIO = 0.5
_EPS = 1e-6


def _max_body(y_ref, m_ref):
    h = pl.program_id(1)
    blk = y_ref[...]
    p = jnp.max(blk, axis=(0, 1, 2))
    prev = jnp.where(h == 0, jnp.full_like(p, -jnp.inf), m_ref[0, 0, :])
    m_ref[0, 0, :] = jnp.maximum(prev, p)


def _sumexp_body(y_ref, m_ref, s_ref, acc_ref, *, nh, wg):
    h = pl.program_id(1)
    blk = y_ref[...]
    m = m_ref[0, 0, :]
    e = jnp.exp(blk - m[None, None, None, :])
    hb, w, c = e.shape[1], e.shape[2], e.shape[3]
    part = jnp.sum(e.reshape(hb * (w // wg), wg, c), axis=0)
    acc_ref[...] = jnp.where(h == 0, part, acc_ref[...] + part)

    @pl.when(h == nh - 1)
    def _():
        s_ref[0, 0, :] = jnp.sum(acc_ref[...], axis=0)


def _ent_body(y_ref, m_ref, s_ref, e_ref, acc_ref, *, nh, wg):
    h = pl.program_id(1)
    blk = y_ref[...]
    m = m_ref[0, 0, :]
    s = s_ref[0, 0, :]
    e = jnp.exp(blk - m[None, None, None, :])
    q = e / s[None, None, None, :] + _EPS
    t = q * jnp.log(q)
    hb, w, c = t.shape[1], t.shape[2], t.shape[3]
    part = jnp.sum(t.reshape(hb * (w // wg), wg, c), axis=0)
    acc_ref[...] = jnp.where(h == 0, part, acc_ref[...] + part)

    @pl.when(h == nh - 1)
    def _():
        e_ref[0, 0, :] = -jnp.sum(acc_ref[...], axis=0)


def _sc_mask_body(ent_hbm, mask_hbm, ent_v, mask_v, *, nb, c, k, lanes):
    cp = 128
    pw = c // cp
    nv = cp // lanes
    wid = lax.axis_index("s") * 2 + lax.axis_index("c")
    b = wid // pw
    base = (wid % pw) * cp

    @pl.when(wid < nb * pw)
    def _():
        pltpu.sync_copy(ent_hbm.at[b, 0], ent_v)

        eis = [ent_v[pl.ds(base + iv * lanes, lanes)] for iv in range(nv)]
        idxs = [lax.iota(jnp.int32, lanes) + (base + iv * lanes)
                for iv in range(nv)]

        one = jnp.full((lanes,), 1, jnp.int32)
        zero = jnp.full((lanes,), 0, jnp.int32)

        def body(jv, cnts):
            start = pl.multiple_of(jv * lanes, lanes)
            vblk = ent_v[pl.ds(start, lanes)]
            out = list(cnts)
            for l in range(lanes):
                vj = jnp.take(vblk, jnp.full((lanes,), l, jnp.int32))
                j = jv * lanes + l
                for iv in range(nv):
                    lt = jnp.where(vj < eis[iv], one, zero)
                    eq = jnp.where(vj == eis[iv], one, zero)
                    jl = jnp.where(j < idxs[iv], one, zero)
                    out[iv] = out[iv] + lt + eq * jl
            return tuple(out)

        cnts = lax.fori_loop(0, c // lanes, body,
                             tuple(jnp.zeros((lanes,), jnp.int32)
                                   for _ in range(nv)))
        for iv in range(nv):
            mask_v[pl.ds(iv * lanes, lanes)] = jnp.where(
                cnts[iv] < k, 1.0, 0.0).astype(jnp.float32)
        pltpu.sync_copy(mask_v, mask_hbm.at[b, 0, pl.ds(base, cp)])


def _mul_body(mask_ref, y_ref, o_ref):
    mk = mask_ref[0, 0, :]
    o_ref[...] = y_ref[...] * mk[None, None, None, :]


def kernel(x):
    B, C, H, W = x.shape
    k = int(C * (1 - _MASK_RATIO))
    y = jnp.transpose(x, (0, 2, 3, 1))
    HB = 28
    NH = H // HB
    WG = 8

    ysp = pl.BlockSpec((1, HB, W, C), lambda b, h: (b, h, 0, 0))
    csp = pl.BlockSpec((1, 1, C), lambda b, h: (b, 0, 0))

    m = pl.pallas_call(
        _max_body,
        grid=(B, NH),
        in_specs=[ysp],
        out_specs=csp,
        out_shape=jax.ShapeDtypeStruct((B, 1, C), jnp.float32),
    )(y)

    s = pl.pallas_call(
        functools.partial(_sumexp_body, nh=NH, wg=WG),
        grid=(B, NH),
        in_specs=[ysp, csp],
        out_specs=csp,
        out_shape=jax.ShapeDtypeStruct((B, 1, C), jnp.float32),
        scratch_shapes=[pltpu.VMEM((WG, C), jnp.float32)],
    )(y, m)

    ent = pl.pallas_call(
        functools.partial(_ent_body, nh=NH, wg=WG),
        grid=(B, NH),
        in_specs=[ysp, csp, csp],
        out_specs=csp,
        out_shape=jax.ShapeDtypeStruct((B, 1, C), jnp.float32),
        scratch_shapes=[pltpu.VMEM((WG, C), jnp.float32)],
    )(y, m, s)

    lanes = 16
    cp = 128
    sc_mask = functools.partial(
        pl.kernel,
        mesh=plsc.VectorSubcoreMesh(core_axis_name="c", subcore_axis_name="s"),
        out_type=jax.ShapeDtypeStruct((B, 1, C), jnp.float32),
        scratch_types=[
            pltpu.VMEM((C,), jnp.float32),
            pltpu.VMEM((cp,), jnp.float32),
        ],
    )(functools.partial(_sc_mask_body, nb=B, c=C, k=k, lanes=lanes))
    mask = sc_mask(ent)

    z = pl.pallas_call(
        _mul_body,
        grid=(B, NH),
        in_specs=[csp, ysp],
        out_specs=ysp,
        out_shape=jax.ShapeDtypeStruct((B, H, W, C), jnp.float32),
    )(mask, y)

    return jnp.transpose(z, (0, 3, 1, 2))
